# baseline (speedup 1.0000x reference)
import functools

import jax
import jax.numpy as jnp
from jax import lax
from jax.experimental import pallas as pl
from jax.experimental.pallas import tpu as pltpu
from jax.experimental.pallas import tpu_sc as plsc

N = 10000
D = 128
E = 320000
NC = 2
NS = 16
NT = NC * NS
NPAD = 10240
C = 80
CPT = 125
ER = E // C
NB = 4
RPT = 320
HALF = 5120
ZR = NPAD // NS
TAIL = N - 31 * RPT

_MESH = plsc.VectorSubcoreMesh(
    core_axis_name="c", subcore_axis_name="s", num_cores=NC, num_subcores=NS)
_SC_PARAMS = pltpu.CompilerParams(needs_layout_passes=False)


def _rsqrt_newton(cv):
  t = cv.astype(jnp.int32)
  e = jnp.zeros((16,), jnp.int32)
  for k in (16, 8, 4, 2, 1):
    m = t >= (1 << k)
    t = jnp.where(m, t >> k, t)
    e = e + jnp.where(m, k, 0)
  y = jnp.exp(e.astype(jnp.float32) * (-0.34657359027997264))
  for _ in range(5):
    y = y * (1.5 - 0.5 * cv * y * y)
  return y


def _k1_body(x_hbm, es_hbm,
             y_hbm, cntd_hbm,
             csrc_sp, cdst_sp,
             idxs_v, idxd_v,
             hs_v, hd_v, ridx_v,
             cl_v, inv_v, xb_v,
             xsem):
  c = lax.axis_index("c")
  s = lax.axis_index("s")
  base = c * HALF + s * RPT
  full = base + RPT <= N
  z16 = jnp.zeros((16,), jnp.float32)

  @pl.when(full)
  def _():
    pltpu.async_copy(x_hbm.at[pl.ds(base, RPT)], xb_v, xsem)

  @pl.when(jnp.logical_not(full))
  def _():
    pltpu.async_copy(x_hbm.at[pl.ds(base, TAIL)], xb_v.at[pl.ds(0, TAIL)],
                     xsem)

  def hz(r, _):
    for k in range(D // 16):
      hs_v[r, pl.ds(16 * k, 16)] = z16
      hd_v[r, pl.ds(16 * k, 16)] = z16
    return 0
  lax.fori_loop(0, NPAD // D, hz, 0)
  iota16 = lax.iota(jnp.int32, 16)
  for j in range(NPAD // D // 16):
    ridx_v[pl.ds(16 * j, 16)] = iota16 + 16 * j
  @pl.when(s < 2)
  def _():
    pltpu.sync_copy(hs_v.at[pl.ds(s * 32, 48)], csrc_sp.at[pl.ds(s * 48, 48)])
    pltpu.sync_copy(hd_v.at[pl.ds(s * 40, 40)], cdst_sp.at[pl.ds(s * 40, 40)])

  pltpu.sync_copy(es_hbm.at[0, s], idxs_v)
  pltpu.sync_copy(es_hbm.at[1, s], idxd_v)
  plsc.subcore_barrier()

  def _hist_vreg(hist, idx):
    cnt, last = plsc.scan_count(idx)
    plsc.addupdate_scatter(hist, [idx >> 7, idx & 127],
                           cnt.astype(jnp.float32), mask=last)

  def csbody(g, _):
    for k in range(C // 16):
      _hist_vreg(hs_v, idxs_v[g, pl.ds(16 * k, 16)])
    return 0
  lax.fori_loop(0, 2 * CPT, csbody, 0)

  def cdbody(g, _):
    for k in range(C // 16):
      _hist_vreg(hd_v, idxd_v[c * CPT + g, pl.ds(16 * k, 16)])
    return 0
  lax.fori_loop(0, CPT, cdbody, 0)

  pltpu.sync_copy(hs_v, csrc_sp.at[ridx_v], add=True)
  pltpu.sync_copy(hd_v, cdst_sp.at[ridx_v], add=True)
  plsc.subcore_barrier()

  r0a = (base >> 7) - ((base >> 7) & 7)
  off0 = base - (r0a << 7)
  pltpu.sync_copy(csrc_sp.at[pl.ds(r0a, 16)], cl_v)
  for i in range(RPT // 16):
    off = off0 + 16 * i
    cv = jnp.maximum(cl_v[off >> 7, pl.ds(off & 127, 16)], 1.0)
    inv_v[pl.ds(16 * i, 16)] = _rsqrt_newton(cv)

  def rbody(r, _):
    iv = plsc.load_gather(inv_v, [jnp.full((16,), r, jnp.int32)])
    for k in range(D // 16):
      xb_v[r, pl.ds(16 * k, 16)] = xb_v[r, pl.ds(16 * k, 16)] * iv
    return 0

  @pl.when(full)
  def _():
    pltpu.make_async_copy(x_hbm.at[pl.ds(base, RPT)], xb_v, xsem).wait()
    lax.fori_loop(0, RPT, rbody, 0)
    pltpu.sync_copy(xb_v, y_hbm.at[pl.ds(base, RPT)])

  @pl.when(jnp.logical_not(full))
  def _():
    pltpu.make_async_copy(x_hbm.at[pl.ds(base, TAIL)], xb_v.at[pl.ds(0, TAIL)],
                          xsem).wait()
    lax.fori_loop(0, TAIL, rbody, 0)
    pltpu.sync_copy(xb_v.at[pl.ds(0, TAIL)], y_hbm.at[pl.ds(base, TAIL)])

  @pl.when(s < 2)
  def _():
    pltpu.sync_copy(cdst_sp.at[pl.ds(s * 40, 40)],
                    cntd_hbm.at[c, pl.ds(s * 40, 40)])


def _k2_body(y_hbm, es_hbm,
             p_hbm,
             agg_sp,
             sidxr, didxr, buf,
             gsem, ssem, isem, idsem):
  c = lax.axis_index("c")
  s = lax.axis_index("s")
  tid = c * NS + s
  grp = tid // 2
  off = (tid % 2) * CPT
  z16 = jnp.zeros((16,), jnp.float32)

  def zb(r, _):
    for k in range(D // 16):
      buf[0, r, pl.ds(16 * k, 16)] = z16
    return 0
  lax.fori_loop(0, C, zb, 0)
  for z in range(ZR // C):
    pltpu.sync_copy(buf.at[0], agg_sp.at[pl.ds(s * ZR + z * C, C)])
  plsc.subcore_barrier()

  for b in range(NB):
    pltpu.sync_copy(es_hbm.at[0, grp, off + b], sidxr.at[b])
    pltpu.sync_copy(es_hbm.at[1, grp, off + b], didxr.at[0, b])
  for b in range(NB):
    pltpu.async_copy(y_hbm.at[sidxr.at[b]], buf.at[b], gsem.at[b])

  RING = CPT - 1
  def mbody(it, _):
    p = it % 2
    for b in range(NB):
      g = it * NB + b
      gn = g + NB
      pltpu.make_async_copy(y_hbm.at[sidxr.at[b]], buf.at[b], gsem.at[b]).wait()
      pltpu.async_copy(buf.at[b], agg_sp.at[didxr.at[p, b]], ssem.at[b],
                       add=True)

      @pl.when(gn < RING)
      def _():
        pltpu.async_copy(es_hbm.at[0, grp, off + gn], sidxr.at[b], isem.at[b])
        pltpu.async_copy(es_hbm.at[1, grp, off + gn], didxr.at[1 - p, b],
                         idsem.at[b])
    for b in range(NB):
      g = it * NB + b
      gn = g + NB
      pltpu.make_async_copy(buf.at[b], agg_sp.at[didxr.at[0, b]],
                            ssem.at[b]).wait()

      @pl.when(gn < RING)
      def _():
        pltpu.make_async_copy(es_hbm.at[0, grp, off], sidxr.at[b],
                              isem.at[b]).wait()
        pltpu.make_async_copy(es_hbm.at[1, grp, off], didxr.at[0, b],
                              idsem.at[b]).wait()
        pltpu.async_copy(y_hbm.at[sidxr.at[b]], buf.at[b], gsem.at[b])
    return 0
  lax.fori_loop(0, RING // NB, mbody, 0)

  pltpu.sync_copy(es_hbm.at[0, grp, off + RING], sidxr.at[0])
  pltpu.sync_copy(es_hbm.at[1, grp, off + RING], didxr.at[0, 0])
  pltpu.async_copy(y_hbm.at[sidxr.at[0]], buf.at[0], gsem.at[0]).wait()
  pltpu.async_copy(buf.at[0], agg_sp.at[didxr.at[0, 0]], ssem.at[0],
                   add=True).wait()
  plsc.subcore_barrier()

  pltpu.sync_copy(agg_sp.at[pl.ds(s * ZR, ZR)], p_hbm.at[c, pl.ds(s * ZR, ZR)])


_k1 = functools.partial(
    pl.kernel,
    out_type=[
        jax.ShapeDtypeStruct((N, D), jnp.float32),
        jax.ShapeDtypeStruct((NC, NPAD // D, D), jnp.float32),
    ],
    mesh=_MESH,
    scratch_types=[
        pltpu.VMEM_SHARED((NPAD // D + 16, D), jnp.float32),
        pltpu.VMEM_SHARED((NPAD // D, D), jnp.float32),
        pltpu.VMEM((2 * CPT, C), jnp.int32),
        pltpu.VMEM((2 * CPT, C), jnp.int32),
        pltpu.VMEM((NPAD // D, D), jnp.float32),
        pltpu.VMEM((NPAD // D, D), jnp.float32),
        pltpu.VMEM((NPAD // D,), jnp.int32),
        pltpu.VMEM((16, D), jnp.float32),
        pltpu.VMEM((RPT,), jnp.float32),
        pltpu.VMEM((RPT, D), jnp.float32),
        pltpu.SemaphoreType.DMA,
    ],
    compiler_params=_SC_PARAMS,
)(_k1_body)

_k2 = functools.partial(
    pl.kernel,
    out_type=jax.ShapeDtypeStruct((NC, NPAD, D), jnp.float32),
    mesh=_MESH,
    scratch_types=[
        pltpu.VMEM_SHARED((NPAD, D), jnp.float32),
        pltpu.VMEM((NB, C), jnp.int32),
        pltpu.VMEM((2, NB, C), jnp.int32),
        pltpu.VMEM((NB, C, D), jnp.float32),
        pltpu.SemaphoreType.DMA((NB,)),
        pltpu.SemaphoreType.DMA((NB,)),
        pltpu.SemaphoreType.DMA((NB,)),
        pltpu.SemaphoreType.DMA((NB,)),
    ],
    compiler_params=_SC_PARAMS,
)(_k2_body)


def _k3_body(p_ref, cnt_ref, w_ref, b_ref, o_ref):
  agg = p_ref[0] + p_ref[1]
  deg = jnp.maximum(cnt_ref[0] + cnt_ref[1], 1.0)
  h = agg * lax.rsqrt(deg)
  o_ref[...] = (
      jnp.dot(h, w_ref[...], preferred_element_type=jnp.float32)
      + b_ref[0:1, :])


_BR = 1000


@jax.jit
def _impl(x, edge_index, W, b):
  b8 = jnp.broadcast_to(b.reshape(1, D), (8, D))

  er = edge_index.reshape(2, NS, 2 * CPT, C)
  y, cntd = _k1(x, er)
  p = _k2(y, er)

  cnt3 = cntd.reshape(NC, NPAD, 1)
  out = pl.pallas_call(
      _k3_body,
      grid=(N // _BR,),
      in_specs=[
          pl.BlockSpec((NC, _BR, D), lambda i: (0, i, 0)),
          pl.BlockSpec((NC, _BR, 1), lambda i: (0, i, 0)),
          pl.BlockSpec((D, D), lambda i: (0, 0)),
          pl.BlockSpec((8, D), lambda i: (0, 0)),
      ],
      out_specs=pl.BlockSpec((_BR, D), lambda i: (i, 0)),
      out_shape=jax.ShapeDtypeStruct((N, D), jnp.float32),
  )(p, cnt3, W, b8)
  return out


def kernel(x, edge_index, W, b):
  return _impl(x, edge_index, W, b)


# speedup vs baseline: 1.1172x; 1.1172x over previous
import functools

import jax
import jax.numpy as jnp
from jax import lax
from jax.experimental import pallas as pl
from jax.experimental.pallas import tpu as pltpu
from jax.experimental.pallas import tpu_sc as plsc

N = 10000
D = 128
E = 320000
NC = 2
NS = 16
NT = NC * NS
NPAD = 10240
C = 80
CPT = 125
ER = E // C
NB = 4
RPT = 320
HALF = 5120
ZR = NPAD // NS
TAIL = N - 31 * RPT

_MESH = plsc.VectorSubcoreMesh(
    core_axis_name="c", subcore_axis_name="s", num_cores=NC, num_subcores=NS)
_SC_PARAMS = pltpu.CompilerParams(needs_layout_passes=False)


def _rsqrt_newton(cv):
  t = cv.astype(jnp.int32)
  e = jnp.zeros((16,), jnp.int32)
  for k in (16, 8, 4, 2, 1):
    m = t >= (1 << k)
    t = jnp.where(m, t >> k, t)
    e = e + jnp.where(m, k, 0)
  y = jnp.exp(e.astype(jnp.float32) * (-0.34657359027997264))
  for _ in range(5):
    y = y * (1.5 - 0.5 * cv * y * y)
  return y


def _k1_body(x_hbm, es_hbm,
             y_hbm, cntd_hbm,
             csrc_sp, cdst_sp,
             idxs_v, idxd_v,
             ones_v, cl_v, inv_v, xb_v,
             csem, xsem):
  c = lax.axis_index("c")
  s = lax.axis_index("s")
  base = c * HALF + s * RPT
  full = base + RPT <= N
  z16 = jnp.zeros((16,), jnp.float32)

  @pl.when(full)
  def _():
    pltpu.async_copy(x_hbm.at[pl.ds(base, RPT)], xb_v, xsem)

  @pl.when(jnp.logical_not(full))
  def _():
    pltpu.async_copy(x_hbm.at[pl.ds(base, TAIL)], xb_v.at[pl.ds(0, TAIL)],
                     xsem)

  for i in range(RPT // 16):
    cl_v[pl.ds(16 * i, 16)] = z16
  for i in range(C // 16):
    ones_v[pl.ds(16 * i, 16)] = z16 + 1.0
  zbase = s * ZR
  pltpu.sync_copy(cl_v, csrc_sp.at[pl.ds(zbase, RPT)])
  pltpu.sync_copy(cl_v, csrc_sp.at[pl.ds(zbase + RPT, RPT)])
  pltpu.sync_copy(cl_v, cdst_sp.at[pl.ds(zbase, RPT)])
  pltpu.sync_copy(cl_v, cdst_sp.at[pl.ds(zbase + RPT, RPT)])

  pltpu.sync_copy(es_hbm.at[0, s], idxs_v)
  pltpu.sync_copy(es_hbm.at[1, s], idxd_v)
  plsc.subcore_barrier()

  def cbody(it, _):
    for j in range(10):
      g = it * 10 + j
      pltpu.async_copy(ones_v, csrc_sp.at[idxs_v.at[g]], csem, add=True)
    for j in range(5):
      g = c * CPT + it * 5 + j
      pltpu.async_copy(ones_v, cdst_sp.at[idxd_v.at[g]], csem, add=True)
    for j in range(15):
      pltpu.make_async_copy(ones_v, csrc_sp.at[idxs_v.at[0]], csem).wait()
    return 0
  lax.fori_loop(0, CPT // 5, cbody, 0)
  plsc.subcore_barrier()

  pltpu.sync_copy(csrc_sp.at[pl.ds(base, RPT)], cl_v)
  for i in range(RPT // 16):
    cv = jnp.maximum(cl_v[pl.ds(16 * i, 16)], 1.0)
    inv_v[pl.ds(16 * i, 16)] = _rsqrt_newton(cv)

  def rbody(r, _):
    iv = plsc.load_gather(inv_v, [jnp.full((16,), r, jnp.int32)])
    for k in range(D // 16):
      xb_v[r, pl.ds(16 * k, 16)] = xb_v[r, pl.ds(16 * k, 16)] * iv
    return 0

  @pl.when(full)
  def _():
    pltpu.make_async_copy(x_hbm.at[pl.ds(base, RPT)], xb_v, xsem).wait()
    lax.fori_loop(0, RPT, rbody, 0)
    pltpu.sync_copy(xb_v, y_hbm.at[pl.ds(base, RPT)])

  @pl.when(jnp.logical_not(full))
  def _():
    pltpu.make_async_copy(x_hbm.at[pl.ds(base, TAIL)], xb_v.at[pl.ds(0, TAIL)],
                          xsem).wait()
    lax.fori_loop(0, TAIL, rbody, 0)
    pltpu.sync_copy(xb_v.at[pl.ds(0, TAIL)], y_hbm.at[pl.ds(base, TAIL)])

  pltpu.sync_copy(cdst_sp.at[pl.ds(s * ZR, ZR)], cntd_hbm.at[c, pl.ds(s * ZR, ZR)])


def _k2_body(y_hbm, es_hbm,
             p_hbm,
             agg_sp,
             sidxr, didxr, buf,
             gsem, ssem, isem, idsem):
  c = lax.axis_index("c")
  s = lax.axis_index("s")
  tid = c * NS + s
  grp = tid // 2
  off = (tid % 2) * CPT
  z16 = jnp.zeros((16,), jnp.float32)

  def zb(r, _):
    for k in range(D // 16):
      buf[0, r, pl.ds(16 * k, 16)] = z16
    return 0
  lax.fori_loop(0, C, zb, 0)
  for z in range(ZR // C):
    pltpu.sync_copy(buf.at[0], agg_sp.at[pl.ds(s * ZR + z * C, C)])
  plsc.subcore_barrier()

  for b in range(NB):
    pltpu.sync_copy(es_hbm.at[0, grp, off + b], sidxr.at[b])
    pltpu.sync_copy(es_hbm.at[1, grp, off + b], didxr.at[0, b])
  for b in range(NB):
    pltpu.async_copy(y_hbm.at[sidxr.at[b]], buf.at[b], gsem.at[b])

  RING = CPT - 1
  def mbody(it, _):
    p = it % 2
    for b in range(NB):
      g = it * NB + b
      gn = g + NB
      pltpu.make_async_copy(y_hbm.at[sidxr.at[b]], buf.at[b], gsem.at[b]).wait()
      pltpu.async_copy(buf.at[b], agg_sp.at[didxr.at[p, b]], ssem.at[b],
                       add=True)

      @pl.when(gn < RING)
      def _():
        pltpu.async_copy(es_hbm.at[0, grp, off + gn], sidxr.at[b], isem.at[b])
        pltpu.async_copy(es_hbm.at[1, grp, off + gn], didxr.at[1 - p, b],
                         idsem.at[b])
    for b in range(NB):
      g = it * NB + b
      gn = g + NB
      pltpu.make_async_copy(buf.at[b], agg_sp.at[didxr.at[0, b]],
                            ssem.at[b]).wait()

      @pl.when(gn < RING)
      def _():
        pltpu.make_async_copy(es_hbm.at[0, grp, off], sidxr.at[b],
                              isem.at[b]).wait()
        pltpu.make_async_copy(es_hbm.at[1, grp, off], didxr.at[0, b],
                              idsem.at[b]).wait()
        pltpu.async_copy(y_hbm.at[sidxr.at[b]], buf.at[b], gsem.at[b])
    return 0
  lax.fori_loop(0, RING // NB, mbody, 0)

  pltpu.sync_copy(es_hbm.at[0, grp, off + RING], sidxr.at[0])
  pltpu.sync_copy(es_hbm.at[1, grp, off + RING], didxr.at[0, 0])
  pltpu.async_copy(y_hbm.at[sidxr.at[0]], buf.at[0], gsem.at[0]).wait()
  pltpu.async_copy(buf.at[0], agg_sp.at[didxr.at[0, 0]], ssem.at[0],
                   add=True).wait()
  plsc.subcore_barrier()

  pltpu.sync_copy(agg_sp.at[pl.ds(s * ZR, ZR)], p_hbm.at[c, pl.ds(s * ZR, ZR)])


_k1 = functools.partial(
    pl.kernel,
    out_type=[
        jax.ShapeDtypeStruct((N, D), jnp.float32),
        jax.ShapeDtypeStruct((NC, NPAD), jnp.float32),
    ],
    mesh=_MESH,
    scratch_types=[
        pltpu.VMEM_SHARED((NPAD,), jnp.float32),
        pltpu.VMEM_SHARED((NPAD,), jnp.float32),
        pltpu.VMEM((2 * CPT, C), jnp.int32),
        pltpu.VMEM((2 * CPT, C), jnp.int32),
        pltpu.VMEM((C,), jnp.float32),
        pltpu.VMEM((RPT,), jnp.float32),
        pltpu.VMEM((RPT,), jnp.float32),
        pltpu.VMEM((RPT, D), jnp.float32),
        pltpu.SemaphoreType.DMA,
        pltpu.SemaphoreType.DMA,
    ],
    compiler_params=_SC_PARAMS,
)(_k1_body)

_k2 = functools.partial(
    pl.kernel,
    out_type=jax.ShapeDtypeStruct((NC, NPAD, D), jnp.float32),
    mesh=_MESH,
    scratch_types=[
        pltpu.VMEM_SHARED((NPAD, D), jnp.float32),
        pltpu.VMEM((NB, C), jnp.int32),
        pltpu.VMEM((2, NB, C), jnp.int32),
        pltpu.VMEM((NB, C, D), jnp.float32),
        pltpu.SemaphoreType.DMA((NB,)),
        pltpu.SemaphoreType.DMA((NB,)),
        pltpu.SemaphoreType.DMA((NB,)),
        pltpu.SemaphoreType.DMA((NB,)),
    ],
    compiler_params=_SC_PARAMS,
)(_k2_body)


def _k3_body(p_ref, cnt_ref, w_ref, b_ref, o_ref):
  agg = p_ref[0] + p_ref[1]
  deg = jnp.maximum(cnt_ref[0] + cnt_ref[1], 1.0)
  h = agg * lax.rsqrt(deg)
  o_ref[...] = (
      jnp.dot(h, w_ref[...], preferred_element_type=jnp.float32)
      + b_ref[0:1, :])


_BR = 1000


@jax.jit
def _impl(x, edge_index, W, b):
  b8 = jnp.broadcast_to(b.reshape(1, D), (8, D))

  er = edge_index.reshape(2, NS, 2 * CPT, C)
  y, cntd = _k1(x, er)
  p = _k2(y, er)

  cnt3 = cntd.reshape(NC, NPAD, 1)
  out = pl.pallas_call(
      _k3_body,
      grid=(N // _BR,),
      in_specs=[
          pl.BlockSpec((NC, _BR, D), lambda i: (0, i, 0)),
          pl.BlockSpec((NC, _BR, 1), lambda i: (0, i, 0)),
          pl.BlockSpec((D, D), lambda i: (0, 0)),
          pl.BlockSpec((8, D), lambda i: (0, 0)),
      ],
      out_specs=pl.BlockSpec((_BR, D), lambda i: (i, 0)),
      out_shape=jax.ShapeDtypeStruct((N, D), jnp.float32),
  )(p, cnt3, W, b8)
  return out


def kernel(x, edge_index, W, b):
  return _impl(x, edge_index, W, b)
